# baseline (speedup 1.0000x reference)
import functools

import jax
import jax.numpy as jnp
from jax import lax
from jax.experimental import pallas as pl
from jax.experimental.pallas import tpu as pltpu
from jax.experimental.pallas import tpu_sc as plsc

NC = 2
NS = 16
NW = NC * NS
WIN = 128
RB = 128

_MESH = plsc.VectorSubcoreMesh(core_axis_name="c", subcore_axis_name="s",
                               num_cores=NC, num_subcores=NS)


DWIN = 128


def _sc_degree(dst3, zeros1, n_pad):
  nwin = dst3.shape[1]
  rpt = n_pad // NS

  @functools.partial(
      pl.kernel,
      out_type=jax.ShapeDtypeStruct((NC * n_pad,), jnp.float32),
      mesh=_MESH,
      scratch_types=[
          pltpu.VMEM((nwin, DWIN), jnp.int32),
          pltpu.VMEM((DWIN,), jnp.float32),
          pltpu.VMEM((rpt,), jnp.float32),
          pltpu.VMEM_SHARED((n_pad,), jnp.float32),
      ],
  )
  def run(dst_hbm, zero_hbm, deg_hbm, didx_all, ones_v, dbuf, deg_sp):
    cid = lax.axis_index("c")
    sid = lax.axis_index("s")
    wid = cid * NS + sid
    pltpu.sync_copy(zero_hbm, dbuf)
    pltpu.sync_copy(dbuf, deg_sp.at[pl.ds(sid * rpt, rpt)])
    for j in range(DWIN // 16):
      ones_v[pl.ds(j * 16, 16)] = jnp.full((16,), 1.0, jnp.float32)
    pltpu.sync_copy(dst_hbm.at[wid], didx_all)
    plsc.subcore_barrier()

    def win(w, carry):
      pltpu.sync_copy(ones_v, deg_sp.at[didx_all.at[w]], add=True)
      return carry

    lax.fori_loop(0, nwin, win, 0)
    plsc.subcore_barrier()
    pltpu.sync_copy(deg_sp.at[pl.ds(sid * rpt, rpt)], dbuf)
    pltpu.sync_copy(dbuf, deg_hbm.at[pl.ds(cid * n_pad + sid * rpt, rpt)])

  return run(dst3, zeros1)


def _sc_scatter_rows(hs, src3, dst3, zrows, n_pad):
  nwin = src3.shape[1]
  h = hs.shape[1]
  rpt = n_pad // NS

  @functools.partial(
      pl.kernel,
      out_type=jax.ShapeDtypeStruct((NC * n_pad, h), jnp.float32),
      mesh=_MESH,
      scratch_types=[
          pltpu.VMEM((nwin, WIN), jnp.int32),
          pltpu.VMEM((nwin, WIN), jnp.int32),
          pltpu.VMEM((WIN, h), jnp.float32),
          pltpu.VMEM_SHARED((n_pad, h), jnp.float32),
      ],
  )
  def run(hs_hbm, src_hbm, dst_hbm, zr_hbm, acc_hbm,
          sidx_all, didx_all, rows, acc_sp):
    rbuf = rows.at[pl.ds(0, RB)]
    cid = lax.axis_index("c")
    sid = lax.axis_index("s")
    wid = cid * NS + sid
    pltpu.sync_copy(zr_hbm, rbuf)
    for k in range(rpt // RB):
      pltpu.sync_copy(rbuf, acc_sp.at[pl.ds(sid * rpt + k * RB, RB)])
    pltpu.sync_copy(src_hbm.at[wid], sidx_all)
    pltpu.sync_copy(dst_hbm.at[wid], didx_all)
    plsc.subcore_barrier()

    def win(w, carry):
      pltpu.sync_copy(hs_hbm.at[sidx_all.at[w]], rows)
      pltpu.sync_copy(rows, acc_sp.at[didx_all.at[w]], add=True)
      return carry

    lax.fori_loop(0, nwin, win, 0)
    plsc.subcore_barrier()
    for k in range(rpt // RB):
      pltpu.sync_copy(acc_sp.at[pl.ds(sid * rpt + k * RB, RB)], rbuf)
      pltpu.sync_copy(
          rbuf, acc_hbm.at[pl.ds(cid * n_pad + sid * rpt + k * RB, RB)])

  return run(hs, src3, dst3, zrows)


def _tc_prep1(x, w1, dega, degb, blk):
  n, f = x.shape
  h = w1.shape[1]
  grid = n // blk

  def body(x_ref, w_ref, da_ref, db_ref, o_ref, dv_ref):
    dv = lax.rsqrt(da_ref[...] + db_ref[...] + 1.0)
    dv_ref[...] = dv
    o_ref[...] = dv * jnp.dot(x_ref[...], w_ref[...],
                              preferred_element_type=jnp.float32)

  return pl.pallas_call(
      body,
      grid=(grid,),
      in_specs=[
          pl.BlockSpec((blk, f), lambda i: (i, 0)),
          pl.BlockSpec((f, h), lambda i: (0, 0)),
          pl.BlockSpec((blk, 1), lambda i: (i, 0)),
          pl.BlockSpec((blk, 1), lambda i: (i, 0)),
      ],
      out_specs=[
          pl.BlockSpec((blk, h), lambda i: (i, 0)),
          pl.BlockSpec((blk, 1), lambda i: (i, 0)),
      ],
      out_shape=[
          jax.ShapeDtypeStruct((n, h), jnp.float32),
          jax.ShapeDtypeStruct((n, 1), jnp.float32),
      ],
  )(x, w1, dega, degb)


def _tc_prep2(acc, hs1, dinv, b1, w2, blk):
  n, h = hs1.shape
  grid = n // blk

  def body(a_ref, hs_ref, d_ref, b_ref, w_ref, o_ref):
    pre = d_ref[...] * (a_ref[0] + a_ref[1] + hs_ref[...]) + b_ref[...]
    x2 = jnp.maximum(pre, 0.0)
    o_ref[...] = d_ref[...] * jnp.dot(x2, w_ref[...],
                                      preferred_element_type=jnp.float32)

  return pl.pallas_call(
      body,
      grid=(grid,),
      in_specs=[
          pl.BlockSpec((NC, blk, h), lambda i: (0, i, 0)),
          pl.BlockSpec((blk, h), lambda i: (i, 0)),
          pl.BlockSpec((blk, 1), lambda i: (i, 0)),
          pl.BlockSpec((1, h), lambda i: (0, 0)),
          pl.BlockSpec((h, h), lambda i: (0, 0)),
      ],
      out_specs=pl.BlockSpec((blk, h), lambda i: (i, 0)),
      out_shape=jax.ShapeDtypeStruct((n, h), jnp.float32),
  )(acc, hs1, dinv, b1, w2)


def _tc_final(acc, hs2, dinv, b2, batch2, wc, bc, num_graphs, blk):
  n, h = hs2.shape
  c = wc.shape[1]
  grid = n // blk

  def body(a_ref, hs_ref, d_ref, b_ref, bt_ref, wc_ref, bc_ref, o_ref, pooled):
    i = pl.program_id(0)

    @pl.when(i == 0)
    def _init():
      pooled[...] = jnp.full((num_graphs, h), -jnp.inf, jnp.float32)

    hx = jnp.maximum(
        d_ref[...] * (a_ref[0] + a_ref[1] + hs_ref[...]) + b_ref[...], 0.0)
    bt = bt_ref[...]
    g_lo = bt[0, 0]
    g_hi = bt[blk - 1, 0]

    def seg(g, carry):
      m = jnp.max(jnp.where(bt == g, hx, -jnp.inf), axis=0, keepdims=True)
      pooled[pl.ds(g, 1), :] = jnp.maximum(pooled[pl.ds(g, 1), :], m)
      return carry

    lax.fori_loop(g_lo, g_hi + 1, seg, 0)

    @pl.when(i == pl.num_programs(0) - 1)
    def _finish():
      logits = jnp.dot(pooled[...], wc_ref[...],
                       preferred_element_type=jnp.float32) + bc_ref[...]
      mx = jnp.max(logits, axis=-1, keepdims=True)
      lse = jnp.log(jnp.sum(jnp.exp(logits - mx), axis=-1, keepdims=True)) + mx
      o_ref[...] = logits - lse

  return pl.pallas_call(
      body,
      grid=(grid,),
      in_specs=[
          pl.BlockSpec((NC, blk, h), lambda i: (0, i, 0)),
          pl.BlockSpec((blk, h), lambda i: (i, 0)),
          pl.BlockSpec((blk, 1), lambda i: (i, 0)),
          pl.BlockSpec((1, h), lambda i: (0, 0)),
          pl.BlockSpec((blk, 1), lambda i: (i, 0)),
          pl.BlockSpec((h, c), lambda i: (0, 0)),
          pl.BlockSpec((1, c), lambda i: (0, 0)),
      ],
      out_specs=pl.BlockSpec((num_graphs, c), lambda i: (0, 0)),
      out_shape=jax.ShapeDtypeStruct((num_graphs, c), jnp.float32),
      scratch_shapes=[pltpu.VMEM((num_graphs, h), jnp.float32)],
  )(acc, hs2, dinv, b2, batch2, wc, bc)


def kernel(x, edge_index, batch, W1, b1, W2, b2, Wc, bc):
  n, f = x.shape
  e = edge_index.shape[1]
  h = W1.shape[1]
  g = 64
  n_pad = ((n + 16 * NS - 1) // (16 * NS)) * (16 * NS)
  assert (n_pad // NS) % RB == 0
  epw = NW * WIN
  e_pad = ((e + epw - 1) // epw) * epw
  ew = e_pad // NW
  nwin = ew // WIN
  blk = 2000
  assert n % blk == 0

  pad_ar = jnp.arange(e_pad - e, dtype=jnp.int32)
  src = jnp.concatenate([edge_index[0], pad_ar % n])
  dst = jnp.concatenate([edge_index[1], n + pad_ar % (n_pad - n)])
  src3 = src.reshape(NW, nwin, WIN)
  dst3 = dst.reshape(NW, nwin, WIN)
  dst3d = dst.reshape(NW, ew // DWIN, DWIN)
  zeros1 = jnp.zeros((n_pad // NS,), jnp.float32)
  zrows = jnp.zeros((RB, h), jnp.float32)

  deg2 = _sc_degree(dst3d, zeros1, n_pad)
  dega = deg2[:n].reshape(n, 1)
  degb = deg2[n_pad:n_pad + n].reshape(n, 1)

  hs1, dinv = _tc_prep1(x, W1, dega, degb, blk)
  acc1 = _sc_scatter_rows(hs1, src3, dst3, zrows, n_pad)
  acc1 = acc1.reshape(NC, n_pad, h)
  hs2 = _tc_prep2(acc1, hs1, dinv, b1.reshape(1, h), W2, blk)
  acc2 = _sc_scatter_rows(hs2, src3, dst3, zrows, n_pad).reshape(NC, n_pad, h)
  return _tc_final(acc2, hs2, dinv, b2.reshape(1, h),
                   batch.reshape(n, 1), Wc, bc.reshape(1, Wc.shape[1]), g, blk)


# speedup vs baseline: 1.0369x; 1.0369x over previous
import functools

import jax
import jax.numpy as jnp
from jax import lax
from jax.experimental import pallas as pl
from jax.experimental.pallas import tpu as pltpu
from jax.experimental.pallas import tpu_sc as plsc

NC = 2
NS = 16
NW = NC * NS
WIN = 128
RB = 128

_MESH = plsc.VectorSubcoreMesh(core_axis_name="c", subcore_axis_name="s",
                               num_cores=NC, num_subcores=NS)


def _sc_degree(dst3, zeros1, n_pad):
  nwin = dst3.shape[1]
  rpt = n_pad // NS

  @functools.partial(
      pl.kernel,
      out_type=jax.ShapeDtypeStruct((NC * n_pad,), jnp.float32),
      mesh=_MESH,
      scratch_types=[
          pltpu.VMEM((nwin, WIN), jnp.int32),
          pltpu.VMEM((WIN,), jnp.float32),
          pltpu.VMEM((rpt,), jnp.float32),
          pltpu.VMEM_SHARED((n_pad,), jnp.float32),
      ],
  )
  def run(dst_hbm, zero_hbm, deg_hbm, didx_all, ones_v, dbuf, deg_sp):
    cid = lax.axis_index("c")
    sid = lax.axis_index("s")
    wid = cid * NS + sid
    pltpu.sync_copy(zero_hbm, dbuf)
    pltpu.sync_copy(dbuf, deg_sp.at[pl.ds(sid * rpt, rpt)])
    for j in range(WIN // 16):
      ones_v[pl.ds(j * 16, 16)] = jnp.full((16,), 1.0, jnp.float32)
    pltpu.sync_copy(dst_hbm.at[wid], didx_all)
    plsc.subcore_barrier()

    def win(w, carry):
      pltpu.sync_copy(ones_v, deg_sp.at[didx_all.at[w]], add=True)
      return carry

    lax.fori_loop(0, nwin, win, 0)
    plsc.subcore_barrier()
    pltpu.sync_copy(deg_sp.at[pl.ds(sid * rpt, rpt)], dbuf)
    pltpu.sync_copy(dbuf, deg_hbm.at[pl.ds(cid * n_pad + sid * rpt, rpt)])

  return run(dst3, zeros1)


def _sc_scatter_rows(hs, src3, dst3, zrows, n_pad):
  nwin = src3.shape[1]
  h = hs.shape[1]
  rpt = n_pad // NS

  @functools.partial(
      pl.kernel,
      out_type=jax.ShapeDtypeStruct((NC * n_pad, h), jnp.float32),
      mesh=_MESH,
      scratch_types=[
          pltpu.VMEM((nwin, WIN), jnp.int32),
          pltpu.VMEM((nwin, WIN), jnp.int32),
          pltpu.VMEM((WIN, h), jnp.float32),
          pltpu.VMEM_SHARED((n_pad, h), jnp.float32),
      ],
  )
  def run(hs_hbm, src_hbm, dst_hbm, zr_hbm, acc_hbm,
          sidx_all, didx_all, rows, acc_sp):
    rbuf = rows
    cid = lax.axis_index("c")
    sid = lax.axis_index("s")
    wid = cid * NS + sid
    pltpu.sync_copy(zr_hbm, rbuf)
    for k in range(rpt // RB):
      pltpu.sync_copy(rbuf, acc_sp.at[pl.ds(sid * rpt + k * RB, RB)])
    pltpu.sync_copy(src_hbm.at[wid], sidx_all)
    pltpu.sync_copy(dst_hbm.at[wid], didx_all)
    plsc.subcore_barrier()

    def win(w, carry):
      pltpu.sync_copy(hs_hbm.at[sidx_all.at[w]], rows)
      pltpu.sync_copy(rows, acc_sp.at[didx_all.at[w]], add=True)
      return carry

    lax.fori_loop(0, nwin, win, 0)
    plsc.subcore_barrier()
    for k in range(rpt // RB):
      pltpu.sync_copy(acc_sp.at[pl.ds(sid * rpt + k * RB, RB)], rbuf)
      pltpu.sync_copy(
          rbuf, acc_hbm.at[pl.ds(cid * n_pad + sid * rpt + k * RB, RB)])

  return run(hs, src3, dst3, zrows)


def _tc_dinv(dega, degb):
  def body(a_ref, b_ref, o_ref):
    o_ref[...] = lax.rsqrt(a_ref[...] + b_ref[...] + 1.0)

  return pl.pallas_call(
      body, out_shape=jax.ShapeDtypeStruct(dega.shape, jnp.float32))(dega, degb)


def _tc_prep1(x, w1, dinv, blk):
  n, f = x.shape
  h = w1.shape[1]
  grid = n // blk

  def body(x_ref, w_ref, d_ref, o_ref):
    o_ref[...] = d_ref[...] * jnp.dot(x_ref[...], w_ref[...],
                                      preferred_element_type=jnp.float32)

  return pl.pallas_call(
      body,
      grid=(grid,),
      in_specs=[
          pl.BlockSpec((blk, f), lambda i: (i, 0)),
          pl.BlockSpec((f, h), lambda i: (0, 0)),
          pl.BlockSpec((blk, 1), lambda i: (i, 0)),
      ],
      out_specs=pl.BlockSpec((blk, h), lambda i: (i, 0)),
      out_shape=jax.ShapeDtypeStruct((n, h), jnp.float32),
  )(x, w1, dinv)


def _tc_prep2(acc, hs1, dinv, b1, w2, blk):
  n, h = hs1.shape
  grid = n // blk

  def body(a_ref, hs_ref, d_ref, b_ref, w_ref, o_ref):
    pre = d_ref[...] * (a_ref[0] + a_ref[1] + hs_ref[...]) + b_ref[...]
    x2 = jnp.maximum(pre, 0.0)
    o_ref[...] = d_ref[...] * jnp.dot(x2, w_ref[...],
                                      preferred_element_type=jnp.float32)

  return pl.pallas_call(
      body,
      grid=(grid,),
      in_specs=[
          pl.BlockSpec((NC, blk, h), lambda i: (0, i, 0)),
          pl.BlockSpec((blk, h), lambda i: (i, 0)),
          pl.BlockSpec((blk, 1), lambda i: (i, 0)),
          pl.BlockSpec((1, h), lambda i: (0, 0)),
          pl.BlockSpec((h, h), lambda i: (0, 0)),
      ],
      out_specs=pl.BlockSpec((blk, h), lambda i: (i, 0)),
      out_shape=jax.ShapeDtypeStruct((n, h), jnp.float32),
  )(acc, hs1, dinv, b1, w2)


def _tc_final(acc, hs2, dinv, b2, batch2, wc, bc, num_graphs, blk):
  n, h = hs2.shape
  c = wc.shape[1]
  grid = n // blk

  def body(a_ref, hs_ref, d_ref, b_ref, bt_ref, wc_ref, bc_ref, o_ref, pooled):
    i = pl.program_id(0)

    @pl.when(i == 0)
    def _init():
      pooled[...] = jnp.full((num_graphs, h), -jnp.inf, jnp.float32)

    hx = jnp.maximum(
        d_ref[...] * (a_ref[0] + a_ref[1] + hs_ref[...]) + b_ref[...], 0.0)
    bt = bt_ref[...]
    sb = blk // 10
    for s0 in range(0, blk, sb):
      bts = bt[s0:s0 + sb]
      hxs = hx[s0:s0 + sb]
      g_lo = bts[0, 0]
      g_hi = bts[sb - 1, 0]

      def seg(g, carry, bts=bts, hxs=hxs):
        m = jnp.max(jnp.where(bts == g, hxs, -jnp.inf), axis=0, keepdims=True)
        pooled[pl.ds(g, 1), :] = jnp.maximum(pooled[pl.ds(g, 1), :], m)
        return carry

      lax.fori_loop(g_lo, g_hi + 1, seg, 0)

    @pl.when(i == pl.num_programs(0) - 1)
    def _finish():
      logits = jnp.dot(pooled[...], wc_ref[...],
                       preferred_element_type=jnp.float32) + bc_ref[...]
      mx = jnp.max(logits, axis=-1, keepdims=True)
      lse = jnp.log(jnp.sum(jnp.exp(logits - mx), axis=-1, keepdims=True)) + mx
      o_ref[...] = logits - lse

  return pl.pallas_call(
      body,
      grid=(grid,),
      in_specs=[
          pl.BlockSpec((NC, blk, h), lambda i: (0, i, 0)),
          pl.BlockSpec((blk, h), lambda i: (i, 0)),
          pl.BlockSpec((blk, 1), lambda i: (i, 0)),
          pl.BlockSpec((1, h), lambda i: (0, 0)),
          pl.BlockSpec((blk, 1), lambda i: (i, 0)),
          pl.BlockSpec((h, c), lambda i: (0, 0)),
          pl.BlockSpec((1, c), lambda i: (0, 0)),
      ],
      out_specs=pl.BlockSpec((num_graphs, c), lambda i: (0, 0)),
      out_shape=jax.ShapeDtypeStruct((num_graphs, c), jnp.float32),
      scratch_shapes=[pltpu.VMEM((num_graphs, h), jnp.float32)],
  )(acc, hs2, dinv, b2, batch2, wc, bc)


def kernel(x, edge_index, batch, W1, b1, W2, b2, Wc, bc):
  n, f = x.shape
  e = edge_index.shape[1]
  h = W1.shape[1]
  g = 64
  n_pad = ((n + 16 * NS - 1) // (16 * NS)) * (16 * NS)
  assert (n_pad // NS) % RB == 0
  epw = NW * WIN
  e_pad = ((e + epw - 1) // epw) * epw
  ew = e_pad // NW
  nwin = ew // WIN
  blk = 2000
  assert n % blk == 0

  pad_ar = jnp.arange(e_pad - e, dtype=jnp.int32)
  src = jnp.concatenate([edge_index[0], pad_ar % n])
  dst = jnp.concatenate([edge_index[1], n + pad_ar % (n_pad - n)])
  src3 = src.reshape(NW, nwin, WIN)
  dst3 = dst.reshape(NW, nwin, WIN)
  zeros1 = jnp.zeros((n_pad // NS,), jnp.float32)
  zrows = jnp.zeros((RB, h), jnp.float32)

  deg2 = _sc_degree(dst3, zeros1, n_pad)
  deg_rows = deg2.reshape(NC, n_pad // 128, 128)
  dinv = _tc_dinv(deg_rows[0], deg_rows[1])
  dinv = dinv.reshape(n_pad, 1)

  hs1 = _tc_prep1(x, W1, dinv, blk)
  acc1 = _sc_scatter_rows(hs1, src3, dst3, zrows, n_pad)
  acc1 = acc1.reshape(NC, n_pad, h)
  hs2 = _tc_prep2(acc1, hs1, dinv, b1.reshape(1, h), W2, blk)
  acc2 = _sc_scatter_rows(hs2, src3, dst3, zrows, n_pad).reshape(NC, n_pad, h)
  return _tc_final(acc2, hs2, dinv, b2.reshape(1, h),
                   batch.reshape(n, 1), Wc, bc.reshape(1, Wc.shape[1]), g, blk)
